# baseline (speedup 1.0000x reference)
import functools

import jax
import jax.numpy as jnp
from jax import lax
from jax.experimental import pallas as pl
from jax.experimental.pallas import tpu as pltpu
from jax.experimental.pallas import tpu_sc as plsc

EPSF = 1e-4
INF = float("inf")
NW = 32
P_BLK = 128


def _rsqrt2(x):
    i = lax.bitcast_convert_type(x, jnp.int32)
    i = 0x5F3759DF - (i >> 1)
    y = lax.bitcast_convert_type(i, jnp.float32)
    y = y * (1.5 - 0.5 * x * y * y)
    y = y * (1.5 - 0.5 * x * y * y)
    return y


def _group(g, p_v, a_v, b_v, v1_v, v2_v, dist_v, grad_v):
    def arow(ref, h):
        return [ref[d, h, pl.ds(g * 16, 16)] for d in range(3)]

    p = [p_v[d, pl.ds(g * 16, 16)] for d in range(3)]

    apb = [None] * 16
    best = None
    bidx = None
    for k in range(16):
        a = arow(a_v, k)
        bk = b_v[k, 0, pl.ds(g * 16, 16)]
        apb[k] = a[0] * p[0] + a[1] * p[1] + a[2] * p[2] - bk
        if k == 0:
            best = apb[0]
            bidx = jnp.zeros((16,), jnp.int32)
        else:
            upd = apb[k] > best
            best = jnp.where(upd, apb[k], best)
            bidx = jnp.where(upd, k, bidx)
    neg = best <= 0.0
    apbe = [apb[k] - EPSF for k in range(16)]

    onz = [None] * 16
    diag = [None] * 16
    for bi in range(4):
        Ai = [arow(a_v, 4 * bi + ii) for ii in range(4)]
        for bj in range(bi, 4):
            Aj = Ai if bj == bi else [arow(a_v, 4 * bj + jj) for jj in range(4)]
            for ii in range(4):
                h = 4 * bi + ii
                for jj in range(4):
                    k = 4 * bj + jj
                    if k < h:
                        continue
                    G = (Ai[ii][0] * Aj[jj][0] + Ai[ii][1] * Aj[jj][1]
                         + Ai[ii][2] * Aj[jj][2])
                    c1 = (apb[h] * G) >= apbe[k]
                    onz[h] = c1 if onz[h] is None else (onz[h] & c1)
                    if k != h:
                        c2 = (apb[k] * G) >= apbe[h]
                        onz[k] = c2 if onz[k] is None else (onz[k] & c2)
                    else:
                        diag[h] = G

    bp = None
    pidx = None
    for h in range(16):
        p2 = apb[h] * apb[h] * diag[h]
        p2 = jnp.where(onz[h], p2, INF)
        if bp is None:
            bp = p2
            pidx = jnp.zeros((16,), jnp.int32)
        else:
            upd = p2 < bp
            bp = jnp.where(upd, p2, bp)
            pidx = jnp.where(upd, h, pidx)

    be = None
    vs = None
    for v in range(16):
        w1 = arow(v1_v, v)
        w2 = arow(v2_v, v)
        dv = [w2[d] - w1[d] for d in range(3)]
        dv2 = dv[0] * dv[0] + dv[1] * dv[1] + dv[2] * dv[2]
        pd = [p[d] - w1[d] for d in range(3)]
        dot = pd[0] * dv[0] + pd[1] * dv[1] + pd[2] * dv[2]
        y = _rsqrt2(dv2)
        inv = jnp.where(dv2 > 0, y * y, 0.0)
        t = jnp.minimum(jnp.maximum(dot * inv, 0.0), 1.0)
        e = [pd[d] - t * dv[d] for d in range(3)]
        e2 = e[0] * e[0] + e[1] * e[1] + e[2] * e[2]
        if be is None:
            be = e2
            vs = e
        else:
            upd = e2 < be
            be = jnp.where(upd, e2, be)
            vs = [jnp.where(upd, e[d], vs[d]) for d in range(3)]

    use_edge = be < bp
    d2 = jnp.where(use_edge, be, bp)
    r2 = _rsqrt2(d2)
    dist_nn = d2 * r2
    rinv = jnp.where(be > 0, r2, 1.0)
    ge = [vs[d] * rinv for d in range(3)]
    dist = jnp.where(neg, best, dist_nn)

    hsel = jnp.where(neg, bidx, pidx)
    colv = lax.iota(jnp.int32, 16) + g * 16
    from_a = neg | jnp.logical_not(use_edge)
    gout = []
    for d in range(3):
        asel = plsc.load_gather(a_v, [jnp.full((16,), d, jnp.int32), hsel, colv])
        gout.append(jnp.where(from_a, asel, ge[d]))

    dist_v[pl.ds(g * 16, 16)] = dist
    for d in range(3):
        grad_v[d, pl.ds(g * 16, 16)] = gout[d]


@functools.cache
def _build(n):
    nfull = n // P_BLK
    tail = n - nfull * P_BLK
    npad = n + (P_BLK - tail) % P_BLK
    assert tail % 16 == 0
    mesh = plsc.VectorSubcoreMesh(core_axis_name="c", subcore_axis_name="s",
                                  num_cores=2, num_subcores=16)

    @functools.partial(
        pl.kernel,
        out_type=(jax.ShapeDtypeStruct((1, npad), jnp.float32),
                  jax.ShapeDtypeStruct((3, npad), jnp.float32)),
        mesh=mesh,
        compiler_params=pltpu.CompilerParams(needs_layout_passes=False,
                                             skip_device_barrier=True),
        scratch_types=[
            [pltpu.VMEM((3, 16, P_BLK), jnp.float32) for _ in range(2)],
            [pltpu.VMEM((16, 1, P_BLK), jnp.float32) for _ in range(2)],
            [pltpu.VMEM((3, 16, P_BLK), jnp.float32) for _ in range(2)],
            [pltpu.VMEM((3, 16, P_BLK), jnp.float32) for _ in range(2)],
            [pltpu.VMEM((3, P_BLK), jnp.float32) for _ in range(2)],
            [pltpu.VMEM((P_BLK,), jnp.float32) for _ in range(2)],
            [pltpu.VMEM((3, P_BLK), jnp.float32) for _ in range(2)],
            [pltpu.SemaphoreType.DMA for _ in range(2)],
            [pltpu.SemaphoreType.DMA for _ in range(2)],
        ],
    )
    def run(p_hbm, a_hbm, b_hbm, v1_hbm, v2_hbm,
            tp_hbm, ta_hbm, tb_hbm, tv1_hbm, tv2_hbm,
            dist_hbm, grad_hbm,
            a_v, b_v, v1_v, v2_v, p_v, dist_v, grad_v, isem, osem):
        c = lax.axis_index("c")
        s = lax.axis_index("s")
        w = s * 2 + c
        nblk_w = (nfull - 1 - w) // NW + 1

        def in_copies(par, base, start):
            mk = pltpu.async_copy if start else pltpu.make_async_copy
            return [
                mk(a_hbm.at[:, :, pl.ds(base, P_BLK)], a_v[par], isem[par]),
                mk(v1_hbm.at[:, :, pl.ds(base, P_BLK)], v1_v[par], isem[par]),
                mk(v2_hbm.at[:, :, pl.ds(base, P_BLK)], v2_v[par], isem[par]),
                mk(b_hbm.at[:, :, pl.ds(base, P_BLK)], b_v[par], isem[par]),
                mk(p_hbm.at[:, pl.ds(base, P_BLK)], p_v[par], isem[par]),
            ]

        def out_copies(par, out_base, start):
            mk = pltpu.async_copy if start else pltpu.make_async_copy
            return [
                mk(dist_v[par], dist_hbm.at[0, pl.ds(out_base, P_BLK)],
                   osem[par]),
                mk(grad_v[par], grad_hbm.at[:, pl.ds(out_base, P_BLK)],
                   osem[par]),
            ]

        def blk_base(j):
            return (w + j * NW) * P_BLK

        def compute(par, ngroups):
            @plsc.parallel_loop(0, ngroups, 1, unroll=2)
            def group_body(g):
                _group(g, p_v[par], a_v[par], b_v[par], v1_v[par], v2_v[par],
                       dist_v[par], grad_v[par])

        @pl.when(nblk_w > 0)
        def _():
            in_copies(0, blk_base(0), True)

        def pair_body(i, carry):
            for par in range(2):
                j = 2 * i + par

                @pl.when(j < nblk_w)
                def _(j=j, par=par):
                    @pl.when(j + 1 < nblk_w)
                    def _():
                        in_copies(1 - par, blk_base(j + 1), True)

                    for cp in in_copies(par, blk_base(j), False):
                        cp.wait()

                    @pl.when(j >= 2)
                    def _():
                        for cp in out_copies(par, blk_base(j - 2), False):
                            cp.wait()

                    compute(par, P_BLK // 16)
                    out_copies(par, blk_base(j), True)
            return carry

        lax.fori_loop(0, (nblk_w + 1) // 2, pair_body, 0)

        for par in range(2):
            @pl.when(nblk_w >= (1 if par == 0 else 2))
            def _(par=par):
                for cp in out_copies(par, 0, False):
                    cp.wait()

        if tail:
            @pl.when(w == NW - 1)
            def _():
                for cp in [
                    pltpu.async_copy(ta_hbm, a_v[0], isem[0]),
                    pltpu.async_copy(tv1_hbm, v1_v[0], isem[0]),
                    pltpu.async_copy(tv2_hbm, v2_v[0], isem[0]),
                    pltpu.async_copy(tb_hbm, b_v[0], isem[0]),
                    pltpu.async_copy(tp_hbm, p_v[0], isem[0]),
                ]:
                    cp.wait()
                compute(0, tail // 16)
                pltpu.sync_copy(dist_v[0],
                                dist_hbm.at[0, pl.ds(nfull * P_BLK, P_BLK)])
                pltpu.sync_copy(grad_v[0],
                                grad_hbm.at[:, pl.ds(nfull * P_BLK, P_BLK)])

    return run


def _pad_cols(x, tail):
    return jnp.pad(x, [(0, 0)] * (x.ndim - 1) + [(0, P_BLK - tail)])


def kernel(point, hyperplane_A, hyperplane_b, v1, v2):
    n = point.shape[0]
    tail = n % P_BLK
    run = _build(n)
    if tail:
        tp = _pad_cols(point[-tail:].transpose(1, 0), tail)
        ta = _pad_cols(hyperplane_A[-tail:].transpose(2, 1, 0), tail)
        tb = _pad_cols(hyperplane_b[-tail:].transpose(1, 2, 0), tail)
        tv1 = _pad_cols(v1[-tail:].transpose(2, 1, 0), tail)
        tv2 = _pad_cols(v2[-tail:].transpose(2, 1, 0), tail)
    else:
        tp = jnp.zeros((3, P_BLK), jnp.float32)
        tb = jnp.zeros((16, 1, P_BLK), jnp.float32)
        ta = tv1 = tv2 = jnp.zeros((3, 16, P_BLK), jnp.float32)
    dist, grad = run(point.transpose(1, 0),
                     hyperplane_A.transpose(2, 1, 0),
                     hyperplane_b.transpose(1, 2, 0),
                     v1.transpose(2, 1, 0),
                     v2.transpose(2, 1, 0),
                     tp, ta, tb, tv1, tv2)
    return dist[0, :n].reshape(n, 1), grad[:, :n].transpose(1, 0)


# speedup vs baseline: 2.6663x; 2.6663x over previous
import functools

import jax
import jax.numpy as jnp
from jax import lax
from jax.experimental import pallas as pl
from jax.experimental.pallas import tpu as pltpu
from jax.experimental.pallas import tpu_sc as plsc

EPSF = 1e-4
INF = float("inf")
NW = 32
P_BLK = 128


def _rsqrt2(x):
    i = lax.bitcast_convert_type(x, jnp.int32)
    i = 0x5F3759DF - (i >> 1)
    y = lax.bitcast_convert_type(i, jnp.float32)
    y = y * (1.5 - 0.5 * x * y * y)
    y = y * (1.5 - 0.5 * x * y * y)
    return y


def _group(g, p_v, a_v, b_v, v1_v, v2_v, dist_v, grad_v):
    def arow(ref, h):
        return [ref[d, h, pl.ds(g * 16, 16)] for d in range(3)]

    p = [p_v[d, pl.ds(g * 16, 16)] for d in range(3)]

    apb = [None] * 16
    best = None
    bidx = None
    for k in range(16):
        a = arow(a_v, k)
        bk = b_v[k, 0, pl.ds(g * 16, 16)]
        apb[k] = a[0] * p[0] + a[1] * p[1] + a[2] * p[2] - bk
        if k == 0:
            best = apb[0]
            bidx = jnp.zeros((16,), jnp.int32)
        else:
            upd = apb[k] > best
            best = jnp.where(upd, apb[k], best)
            bidx = jnp.where(upd, k, bidx)
    neg = best <= 0.0
    apbe = [apb[k] - EPSF for k in range(16)]

    onz = [None] * 16
    diag = [None] * 16
    for bi in range(4):
        Ai = [arow(a_v, 4 * bi + ii) for ii in range(4)]
        for bj in range(bi, 4):
            Aj = Ai if bj == bi else [arow(a_v, 4 * bj + jj) for jj in range(4)]
            for ii in range(4):
                h = 4 * bi + ii
                for jj in range(4):
                    k = 4 * bj + jj
                    if k < h:
                        continue
                    G = (Ai[ii][0] * Aj[jj][0] + Ai[ii][1] * Aj[jj][1]
                         + Ai[ii][2] * Aj[jj][2])
                    c1 = (apb[h] * G) >= apbe[k]
                    onz[h] = c1 if onz[h] is None else (onz[h] & c1)
                    if k != h:
                        c2 = (apb[k] * G) >= apbe[h]
                        onz[k] = c2 if onz[k] is None else (onz[k] & c2)
                    else:
                        diag[h] = G

    bp = None
    pidx = None
    for h in range(16):
        p2 = apb[h] * apb[h] * diag[h]
        p2 = jnp.where(onz[h], p2, INF)
        if bp is None:
            bp = p2
            pidx = jnp.zeros((16,), jnp.int32)
        else:
            upd = p2 < bp
            bp = jnp.where(upd, p2, bp)
            pidx = jnp.where(upd, h, pidx)

    be = None
    vs = None
    for v in range(16):
        w1 = arow(v1_v, v)
        w2 = arow(v2_v, v)
        dv = [w2[d] - w1[d] for d in range(3)]
        dv2 = dv[0] * dv[0] + dv[1] * dv[1] + dv[2] * dv[2]
        pd = [p[d] - w1[d] for d in range(3)]
        dot = pd[0] * dv[0] + pd[1] * dv[1] + pd[2] * dv[2]
        y = _rsqrt2(dv2)
        inv = jnp.where(dv2 > 0, y * y, 0.0)
        t = jnp.minimum(jnp.maximum(dot * inv, 0.0), 1.0)
        e = [pd[d] - t * dv[d] for d in range(3)]
        e2 = e[0] * e[0] + e[1] * e[1] + e[2] * e[2]
        if be is None:
            be = e2
            vs = e
        else:
            upd = e2 < be
            be = jnp.where(upd, e2, be)
            vs = [jnp.where(upd, e[d], vs[d]) for d in range(3)]

    use_edge = be < bp
    d2 = jnp.where(use_edge, be, bp)
    r2 = _rsqrt2(d2)
    dist_nn = d2 * r2
    rinv = jnp.where(be > 0, r2, 1.0)
    ge = [vs[d] * rinv for d in range(3)]
    dist = jnp.where(neg, best, dist_nn)

    hsel = jnp.where(neg, bidx, pidx)
    colv = lax.iota(jnp.int32, 16) + g * 16
    from_a = neg | jnp.logical_not(use_edge)
    gout = []
    for d in range(3):
        asel = plsc.load_gather(a_v, [jnp.full((16,), d, jnp.int32), hsel, colv])
        gout.append(jnp.where(from_a, asel, ge[d]))

    dist_v[pl.ds(g * 16, 16)] = dist
    for d in range(3):
        grad_v[d, pl.ds(g * 16, 16)] = gout[d]


@functools.cache
def _build(n):
    nfull = n // P_BLK
    tail = n - nfull * P_BLK
    npad = n + (P_BLK - tail) % P_BLK
    assert tail % 16 == 0
    mesh = plsc.VectorSubcoreMesh(core_axis_name="c", subcore_axis_name="s",
                                  num_cores=2, num_subcores=16)

    @functools.partial(
        pl.kernel,
        out_type=(jax.ShapeDtypeStruct((1, npad), jnp.float32),
                  jax.ShapeDtypeStruct((3, npad), jnp.float32)),
        mesh=mesh,
        compiler_params=pltpu.CompilerParams(needs_layout_passes=False),
        scratch_types=[
            [pltpu.VMEM((3, 16, P_BLK), jnp.float32) for _ in range(2)],
            [pltpu.VMEM((16, 1, P_BLK), jnp.float32) for _ in range(2)],
            [pltpu.VMEM((3, 16, P_BLK), jnp.float32) for _ in range(2)],
            [pltpu.VMEM((3, 16, P_BLK), jnp.float32) for _ in range(2)],
            [pltpu.VMEM((3, P_BLK), jnp.float32) for _ in range(2)],
            [pltpu.VMEM((P_BLK,), jnp.float32) for _ in range(2)],
            [pltpu.VMEM((3, P_BLK), jnp.float32) for _ in range(2)],
            [pltpu.SemaphoreType.DMA for _ in range(2)],
            [pltpu.SemaphoreType.DMA for _ in range(2)],
        ],
    )
    def run(p_hbm, a_hbm, b_hbm, v1_hbm, v2_hbm,
            tp_hbm, ta_hbm, tb_hbm, tv1_hbm, tv2_hbm,
            dist_hbm, grad_hbm,
            a_v, b_v, v1_v, v2_v, p_v, dist_v, grad_v, isem, osem):
        c = lax.axis_index("c")
        s = lax.axis_index("s")
        w = s * 2 + c
        nblk_w = (nfull - 1 - w) // NW + 1

        def in_copies(par, base, start):
            mk = pltpu.async_copy if start else pltpu.make_async_copy
            return [
                mk(a_hbm.at[:, :, pl.ds(base, P_BLK)], a_v[par], isem[par]),
                mk(v1_hbm.at[:, :, pl.ds(base, P_BLK)], v1_v[par], isem[par]),
                mk(v2_hbm.at[:, :, pl.ds(base, P_BLK)], v2_v[par], isem[par]),
                mk(b_hbm.at[:, :, pl.ds(base, P_BLK)], b_v[par], isem[par]),
                mk(p_hbm.at[:, pl.ds(base, P_BLK)], p_v[par], isem[par]),
            ]

        def out_copies(par, out_base, start):
            mk = pltpu.async_copy if start else pltpu.make_async_copy
            return [
                mk(dist_v[par], dist_hbm.at[0, pl.ds(out_base, P_BLK)],
                   osem[par]),
                mk(grad_v[par], grad_hbm.at[:, pl.ds(out_base, P_BLK)],
                   osem[par]),
            ]

        def blk_base(j):
            return (w + j * NW) * P_BLK

        def compute(par, ngroups):
            def group_body(g, gc):
                _group(g, p_v[par], a_v[par], b_v[par], v1_v[par], v2_v[par],
                       dist_v[par], grad_v[par])
                return gc

            lax.fori_loop(0, ngroups, group_body, 0)

        @pl.when(nblk_w > 0)
        def _():
            in_copies(0, blk_base(0), True)

        def pair_body(i, carry):
            for par in range(2):
                j = 2 * i + par

                @pl.when(j < nblk_w)
                def _(j=j, par=par):
                    @pl.when(j + 1 < nblk_w)
                    def _():
                        in_copies(1 - par, blk_base(j + 1), True)

                    for cp in in_copies(par, blk_base(j), False):
                        cp.wait()

                    @pl.when(j >= 2)
                    def _():
                        for cp in out_copies(par, blk_base(j - 2), False):
                            cp.wait()

                    compute(par, P_BLK // 16)
                    out_copies(par, blk_base(j), True)
            return carry

        lax.fori_loop(0, (nblk_w + 1) // 2, pair_body, 0)

        for par in range(2):
            @pl.when(nblk_w >= (1 if par == 0 else 2))
            def _(par=par):
                for cp in out_copies(par, 0, False):
                    cp.wait()

        if tail:
            @pl.when(w == NW - 1)
            def _():
                for cp in [
                    pltpu.async_copy(ta_hbm, a_v[0], isem[0]),
                    pltpu.async_copy(tv1_hbm, v1_v[0], isem[0]),
                    pltpu.async_copy(tv2_hbm, v2_v[0], isem[0]),
                    pltpu.async_copy(tb_hbm, b_v[0], isem[0]),
                    pltpu.async_copy(tp_hbm, p_v[0], isem[0]),
                ]:
                    cp.wait()
                compute(0, tail // 16)
                pltpu.sync_copy(dist_v[0],
                                dist_hbm.at[0, pl.ds(nfull * P_BLK, P_BLK)])
                pltpu.sync_copy(grad_v[0],
                                grad_hbm.at[:, pl.ds(nfull * P_BLK, P_BLK)])

    return run


def _pad_cols(x, tail):
    return jnp.pad(x, [(0, 0)] * (x.ndim - 1) + [(0, P_BLK - tail)])


def kernel(point, hyperplane_A, hyperplane_b, v1, v2):
    n = point.shape[0]
    tail = n % P_BLK
    run = _build(n)
    if tail:
        tp = _pad_cols(point[-tail:].transpose(1, 0), tail)
        ta = _pad_cols(hyperplane_A[-tail:].transpose(2, 1, 0), tail)
        tb = _pad_cols(hyperplane_b[-tail:].transpose(1, 2, 0), tail)
        tv1 = _pad_cols(v1[-tail:].transpose(2, 1, 0), tail)
        tv2 = _pad_cols(v2[-tail:].transpose(2, 1, 0), tail)
    else:
        tp = jnp.zeros((3, P_BLK), jnp.float32)
        tb = jnp.zeros((16, 1, P_BLK), jnp.float32)
        ta = tv1 = tv2 = jnp.zeros((3, 16, P_BLK), jnp.float32)
    dist, grad = run(point.transpose(1, 0),
                     hyperplane_A.transpose(2, 1, 0),
                     hyperplane_b.transpose(1, 2, 0),
                     v1.transpose(2, 1, 0),
                     v2.transpose(2, 1, 0),
                     tp, ta, tb, tv1, tv2)
    return dist[0, :n].reshape(n, 1), grad[:, :n].transpose(1, 0)
